# baseline (speedup 1.0000x reference)
import jax
import jax.numpy as jnp
from jax import lax
from jax.experimental import pallas as pl
from jax.experimental.pallas import tpu as pltpu
from jax.experimental.pallas import tpu_sc as plsc

B = 1024
L = 200
D = 768
N = B * L
NC = 2
NS = 16
NW = NC * NS
LANES = 16
LB = 8
NLB = L // LB
GB = 4
W = GB * LB
BPW = B // NW
CPS = BPW // GB
CHUNKS = NLB * CPS
NSETS = 4


def _prep_tc_body(x_ref, sl_ref, ti_out, sl_out):
    xb = x_ref[...].reshape(B, NLB, LB)
    sb = sl_ref[...].reshape(B, NLB, LB)
    ti_out[...] = jnp.transpose(xb, (1, 0, 2))
    sl_out[...] = jnp.transpose(sb, (1, 0, 2))


def _prep_blocked(x, sl):
    return pl.pallas_call(
        _prep_tc_body,
        out_shape=[
            jax.ShapeDtypeStruct((NLB, B, LB), jnp.int32),
            jax.ShapeDtypeStruct((NLB, B, LB), jnp.int32),
        ],
    )(x, sl)


def _sc_body(tok_hbm, seg_hbm, pe_hbm, ti_hbm, sl_hbm, out_hbm,
             comb_sm, seg_v,
             ti0, sl0, tok0, ti1, sl1, tok1,
             ti2, sl2, tok2, ti3, sl3, tok3,
             sem_t0, sem_w0, sem_t1, sem_w1,
             sem_t2, sem_w2, sem_t3, sem_w3):
    wid = lax.axis_index("s") * NC + lax.axis_index("c")
    b0 = wid * BPW

    pltpu.sync_copy(seg_hbm, seg_v)

    sets = (
        (ti0, sl0, tok0, sem_t0, sem_w0),
        (ti1, sl1, tok1, sem_t1, sem_w1),
        (ti2, sl2, tok2, sem_t2, sem_w2),
        (ti3, sl3, tok3, sem_t3, sem_w3),
    )

    def idx_off(jj):
        g = jj // CPS
        jb = lax.rem(jj, CPS)
        return g * (B * LB) + b0 * LB + jb * W

    def issue(jj, p):
        ti_v, sl_v, tok_v, sem_t, _ = sets[p]
        off = idx_off(jj)
        pltpu.sync_copy(ti_hbm.at[pl.ds(off, W)], ti_v)
        pltpu.sync_copy(sl_hbm.at[pl.ds(off, W)], sl_v.at[pl.ds(0, W)])
        pltpu.async_copy(tok_hbm.at[ti_v], tok_v, sem_t)

    def wait_gather(p):
        ti_v, _, tok_v, sem_t, _ = sets[p]
        pltpu.make_async_copy(tok_hbm.at[ti_v], tok_v, sem_t).wait()

    def rebuild_comb(g):
        for s in range(3):
            pltpu.sync_copy(pe_hbm.at[pl.ds(g * LB, LB)], comb_sm.at[s])

        @plsc.parallel_loop(0, LB)
        def _row(r):
            for s in range(3):
                for c in range(0, D, LANES):
                    comb_sm.at[s, r, pl.ds(c, LANES)][...] = (
                        comb_sm.at[s, r, pl.ds(c, LANES)][...]
                        + seg_v.at[s, pl.ds(c, LANES)][...])

    def add(p):
        _, sl_v, tok_v, _, _ = sets[p]

        @plsc.parallel_loop(0, W, unroll=2)
        def _row(r):
            s = sl_v.at[pl.ds(r, LANES)][...][0]
            ll = lax.rem(r, LB)
            for c in range(0, D, LANES):
                tok_v.at[r, pl.ds(c, LANES)][...] = (
                    tok_v.at[r, pl.ds(c, LANES)][...]
                    + comb_sm.at[s, ll, pl.ds(c, LANES)][...])

    def out_row(jj, k):
        g = jj // CPS
        jb = lax.rem(jj, CPS)
        return (b0 + jb * GB + k) * L + g * LB

    def start_writes(jj, p):
        _, _, tok_v, _, sem_w = sets[p]
        for k in range(GB):
            pltpu.async_copy(
                tok_v.at[pl.ds(k * LB, LB)],
                out_hbm.at[pl.ds(out_row(jj, k), LB)], sem_w)

    def wait_writes(jj, p):
        _, _, tok_v, _, sem_w = sets[p]
        for k in range(GB):
            pltpu.make_async_copy(
                tok_v.at[pl.ds(k * LB, LB)],
                out_hbm.at[pl.ds(out_row(jj, k), LB)], sem_w).wait()

    issue(0, 0)
    issue(1, 1)
    issue(2, 2)

    @pl.loop(0, CHUNKS, step=NSETS)
    def _chunk(j):
        for p in range(NSETS):
            jj = j + p
            if p == 0:
                @pl.when(lax.rem(j, CPS) == 0)
                def _():
                    rebuild_comb(j // CPS)

            wait_gather(p)
            add(p)
            start_writes(jj, p)

            q = (p + 3) % NSETS

            @pl.when(jj + 3 < CHUNKS)
            def _():
                @pl.when(jj >= 1)
                def _():
                    wait_writes(jj - 1, q)

                issue(jj + 3, q)

    for p in range(NSETS):
        wait_writes(CHUNKS - NSETS + p, p)


def kernel(x, segment_label, token_weight, segment_weight, pe):
    ti_blk, sl_blk = _prep_blocked(
        x.astype(jnp.int32), segment_label.astype(jnp.int32))
    pe_l = pe[0, :L]

    mesh = plsc.VectorSubcoreMesh(core_axis_name="c", subcore_axis_name="s")
    sc = pl.kernel(
        _sc_body,
        out_type=jax.ShapeDtypeStruct((N, D), jnp.float32),
        mesh=mesh,
        scratch_types=[
            pltpu.VMEM((3, LB, D), jnp.float32),
            pltpu.VMEM((3, D), jnp.float32),
            pltpu.VMEM((W,), jnp.int32),
            pltpu.VMEM((W + LANES,), jnp.int32),
            pltpu.VMEM((W, D), jnp.float32),
            pltpu.VMEM((W,), jnp.int32),
            pltpu.VMEM((W + LANES,), jnp.int32),
            pltpu.VMEM((W, D), jnp.float32),
            pltpu.VMEM((W,), jnp.int32),
            pltpu.VMEM((W + LANES,), jnp.int32),
            pltpu.VMEM((W, D), jnp.float32),
            pltpu.VMEM((W,), jnp.int32),
            pltpu.VMEM((W + LANES,), jnp.int32),
            pltpu.VMEM((W, D), jnp.float32),
            pltpu.SemaphoreType.DMA,
            pltpu.SemaphoreType.DMA,
            pltpu.SemaphoreType.DMA,
            pltpu.SemaphoreType.DMA,
            pltpu.SemaphoreType.DMA,
            pltpu.SemaphoreType.DMA,
            pltpu.SemaphoreType.DMA,
            pltpu.SemaphoreType.DMA,
        ],
    )
    out = sc(token_weight, segment_weight, pe_l,
             ti_blk.reshape(-1), sl_blk.reshape(-1))
    return out.reshape(B, L, D)


# speedup vs baseline: 1.2640x; 1.2640x over previous
import jax
import jax.numpy as jnp
from jax import lax
from jax.experimental import pallas as pl
from jax.experimental.pallas import tpu as pltpu
from jax.experimental.pallas import tpu_sc as plsc

B = 1024
L = 200
D = 768
N = B * L
NC = 2
NS = 16
NW = NC * NS
LANES = 16
ROWS_PER_W = N // NW
W = 32
BLK = 1600
NBLK = ROWS_PER_W // BLK
CHUNKS = BLK // W


def _comb_tc_body(pe_ref, seg_ref, out_ref):
    pe = pe_ref[...]
    seg = seg_ref[...]
    out_ref[...] = (pe[:, None, :] + seg[None, :, :]).reshape(L * 3, D)


def _build_comb(pe_l, seg_w):
    return pl.pallas_call(
        _comb_tc_body,
        out_shape=jax.ShapeDtypeStruct((L * 3, D), jnp.float32),
    )(pe_l, seg_w)


def _sc_body(tok_hbm, comb_hbm, ti_hbm, sl_hbm, out_hbm,
             ti_all, ci_all,
             tok0, comb0, tok1, comb1,
             sem_t0, sem_c0, sem_w0, sem_t1, sem_c1, sem_w1):
    wid = lax.axis_index("s") * NC + lax.axis_index("c")
    base = wid * ROWS_PER_W

    sets = (
        (tok0, comb0, sem_t0, sem_c0, sem_w0),
        (tok1, comb1, sem_t1, sem_c1, sem_w1),
    )

    @pl.loop(0, NBLK)
    def _block(bk):
        blk_base = base + bk * BLK

        pltpu.sync_copy(ti_hbm.at[pl.ds(blk_base, BLK)], ti_all)
        pltpu.sync_copy(sl_hbm.at[pl.ds(blk_base, BLK)], ci_all)

        @plsc.parallel_loop(0, BLK, step=LANES, unroll=4)
        def _ci(v):
            flat = blk_base + v + lax.iota(jnp.int32, LANES)
            s = ci_all.at[pl.ds(v, LANES)][...]
            ci_all.at[pl.ds(v, LANES)][...] = lax.rem(flat, L) * 3 + s

        def issue(j, p):
            tok_v, comb_v, sem_t, sem_c, _ = sets[p]
            off = j * W
            pltpu.async_copy(
                tok_hbm.at[ti_all.at[pl.ds(off, W)]], tok_v, sem_t)
            pltpu.async_copy(
                comb_hbm.at[ci_all.at[pl.ds(off, W)]], comb_v, sem_c)

        def wait_gathers(j, p):
            tok_v, comb_v, sem_t, sem_c, _ = sets[p]
            off = j * W
            pltpu.make_async_copy(
                tok_hbm.at[ti_all.at[pl.ds(off, W)]], tok_v, sem_t).wait()
            pltpu.make_async_copy(
                comb_hbm.at[ci_all.at[pl.ds(off, W)]], comb_v, sem_c).wait()

        def add(p):
            tok_v, comb_v, _, _, _ = sets[p]

            @plsc.parallel_loop(0, W, unroll=2)
            def _row(r):
                for c in range(0, D, LANES):
                    tok_v.at[r, pl.ds(c, LANES)][...] = (
                        tok_v.at[r, pl.ds(c, LANES)][...]
                        + comb_v.at[r, pl.ds(c, LANES)][...])

        def start_write(j, p):
            tok_v, _, _, _, sem_w = sets[p]
            pltpu.async_copy(
                tok_v, out_hbm.at[pl.ds(blk_base + j * W, W)], sem_w)

        def wait_write(j, p):
            tok_v, _, _, _, sem_w = sets[p]
            pltpu.make_async_copy(
                tok_v, out_hbm.at[pl.ds(blk_base + j * W, W)], sem_w).wait()

        issue(0, 0)

        @pl.loop(0, CHUNKS, step=2)
        def _chunk(j):
            @pl.when(j > 0)
            def _():
                wait_write(j - 1, 1)

            issue(j + 1, 1)
            wait_gathers(j, 0)
            add(0)
            start_write(j, 0)
            wait_gathers(j + 1, 1)
            add(1)
            wait_write(j, 0)

            @pl.when(j + 2 < CHUNKS)
            def _():
                issue(j + 2, 0)

            start_write(j + 1, 1)

        wait_write(CHUNKS - 1, 1)


def kernel(x, segment_label, token_weight, segment_weight, pe):
    ti = x.reshape(N).astype(jnp.int32)
    sl = segment_label.reshape(N).astype(jnp.int32)
    comb = _build_comb(pe[0, :L], segment_weight)

    mesh = plsc.VectorSubcoreMesh(core_axis_name="c", subcore_axis_name="s")
    sc = pl.kernel(
        _sc_body,
        out_type=jax.ShapeDtypeStruct((N, D), jnp.float32),
        mesh=mesh,
        scratch_types=[
            pltpu.VMEM((BLK,), jnp.int32),
            pltpu.VMEM((BLK,), jnp.int32),
            pltpu.VMEM((W, D), jnp.float32),
            pltpu.VMEM((W, D), jnp.float32),
            pltpu.VMEM((W, D), jnp.float32),
            pltpu.VMEM((W, D), jnp.float32),
            pltpu.SemaphoreType.DMA,
            pltpu.SemaphoreType.DMA,
            pltpu.SemaphoreType.DMA,
            pltpu.SemaphoreType.DMA,
            pltpu.SemaphoreType.DMA,
            pltpu.SemaphoreType.DMA,
        ],
    )
    out = sc(token_weight, comb, ti, sl)
    return out.reshape(B, L, D)
